# baseline (speedup 1.0000x reference)
import functools

import jax
import jax.numpy as jnp
from jax import lax
from jax.experimental import pallas as pl
from jax.experimental.pallas import tpu as pltpu
from jax.experimental.pallas import tpu_sc as plsc

B = 16384
D = 32
F = 16
NC = 2
NS = 16
L = 16
NW = NC * NS
BPW = B // NW
CHUNK = 128
NCHUNK = BPW // CHUNK
NG = BPW // L

_mesh = plsc.VectorSubcoreMesh(
    core_axis_name="c", subcore_axis_name="s", num_cores=NC, num_subcores=NS
)


def _body(u_hbm, i_hbm, p_hbm, q_hbm, bu_hbm, bi_hbm, mu16_hbm, fw_hbm,
          feat_hbm, out_hbm, uv, iv, ubh, ibh, pv, qv, fv, buv, biv, fwv,
          muv, outv, sem):
    wid = lax.axis_index("s") * NC + lax.axis_index("c")
    base = pl.multiple_of(wid * BPW, BPW)

    pltpu.sync_copy(u_hbm.at[pl.ds(base, BPW)], uv)
    pltpu.sync_copy(i_hbm.at[pl.ds(base, BPW)], iv)
    pltpu.sync_copy(fw_hbm, fwv)
    pltpu.sync_copy(mu16_hbm, muv)

    for t in range(BPW // L):
        s = pl.ds(t * L, L)
        ubh[s] = jax.lax.shift_right_logical(uv[s], 4)
        ibh[s] = jax.lax.shift_right_logical(iv[s], 4)

    cps = []
    for j in range(NCHUNK):
        r = pl.ds(j * CHUNK, CHUNK)
        cps.append(pltpu.async_copy(p_hbm.at[uv.at[r]], pv.at[r], sem))
        cps.append(pltpu.async_copy(q_hbm.at[iv.at[r]], qv.at[r], sem))
        cps.append(pltpu.async_copy(feat_hbm.at[iv.at[r]], fv.at[r], sem))
        cps.append(pltpu.async_copy(bu_hbm.at[ubh.at[r]], buv.at[r], sem))
        cps.append(pltpu.async_copy(bi_hbm.at[ibh.at[r]], biv.at[r], sem))
    for c in cps:
        c.wait()

    lane = lax.iota(jnp.int32, L)
    mu_vec = muv[...]
    mask15 = jnp.full((L,), 15, jnp.int32)

    def group(g, carry):
        row0 = pl.multiple_of(g * L, L)
        ridx = row0 + lane
        uvec = uv[pl.ds(row0, L)]
        ivec = iv[pl.ds(row0, L)]
        bu_g = plsc.load_gather(buv, [ridx, uvec & mask15])
        bi_g = plsc.load_gather(biv, [ridx, ivec & mask15])
        feats = [plsc.load_gather(fv, [ridx, jnp.full((L,), f, jnp.int32)])
                 for f in range(F)]
        acc = bu_g + bi_g + mu_vec
        for d in range(D):
            dd = jnp.full((L,), d, jnp.int32)
            p_d = plsc.load_gather(pv, [ridx, dd])
            q_d = plsc.load_gather(qv, [ridx, dd])
            fwd = fwv[d, :]
            for f in range(F):
                q_d = q_d + feats[f] * fwd[f]
            acc = acc + p_d * q_d
        outv[pl.ds(row0, L)] = acc
        return carry

    lax.fori_loop(0, NG, group, 0)
    pltpu.sync_copy(outv, out_hbm.at[pl.ds(base, BPW)])


_hybrid_mf_sc = functools.partial(
    pl.kernel,
    out_type=jax.ShapeDtypeStruct((B,), jnp.float32),
    mesh=_mesh,
    scratch_types=[
        pltpu.VMEM((BPW,), jnp.int32),
        pltpu.VMEM((BPW,), jnp.int32),
        pltpu.VMEM((BPW,), jnp.int32),
        pltpu.VMEM((BPW,), jnp.int32),
        pltpu.VMEM((BPW, D), jnp.float32),
        pltpu.VMEM((BPW, D), jnp.float32),
        pltpu.VMEM((BPW, F), jnp.float32),
        pltpu.VMEM((BPW, L), jnp.float32),
        pltpu.VMEM((BPW, L), jnp.float32),
        pltpu.VMEM((D, F), jnp.float32),
        pltpu.VMEM((L,), jnp.float32),
        pltpu.VMEM((BPW,), jnp.float32),
        pltpu.SemaphoreType.DMA,
    ],
    compiler_params=pltpu.CompilerParams(
        needs_layout_passes=False, use_tc_tiling_on_sc=False
    ),
)(_body)


def kernel(u, i, P, Q, bu, bi, mu, F_w, item_features):
    nu = P.shape[0]
    ni = Q.shape[0]
    mu16 = jnp.broadcast_to(mu.astype(jnp.float32), (L,))
    return _hybrid_mf_sc(
        u.astype(jnp.int32), i.astype(jnp.int32),
        P, Q,
        bu.reshape(nu // L, L), bi.reshape(ni // L, L),
        mu16, F_w, item_features,
    )
